# baseline (speedup 1.0000x reference)
import functools

import jax
import jax.numpy as jnp
from jax import lax
from jax.experimental import pallas as pl
from jax.experimental.pallas import tpu as pltpu
from jax.experimental.pallas import tpu_sc as plsc

B = 4096
S = 200
D = 64
NW = 32
ROWS_PER_W = B // NW
SUP = 2
NSUP = ROWS_PER_W // SUP
SPLITS = ((0, 104), (104, 96))

_mesh = plsc.VectorSubcoreMesh(core_axis_name="c", subcore_axis_name="s")


@functools.partial(
    pl.kernel,
    out_type=jax.ShapeDtypeStruct((B, S, D), jnp.float32),
    mesh=_mesh,
    scratch_types=[
        pltpu.VMEM((ROWS_PER_W, S), jnp.int32),
        pltpu.VMEM((2, SUP, S, D), jnp.float32),
        pltpu.SemaphoreType.DMA,
        pltpu.SemaphoreType.DMA,
    ],
    compiler_params=pltpu.CompilerParams(use_tc_tiling_on_sc=False),
)
def _gather_kernel(x_hbm, table_hbm, out_hbm, idx_v, rows_v, gsem, wsem):
    wid = lax.axis_index("s") * 2 + lax.axis_index("c")
    row_base = wid * ROWS_PER_W
    pltpu.sync_copy(x_hbm.at[pl.ds(row_base, ROWS_PER_W)], idx_v)

    def fire_gathers(sc, buf):
        cps = []
        for s in range(SUP):
            for off, ln in SPLITS:
                cps.append(
                    pltpu.async_copy(
                        table_hbm.at[idx_v.at[sc * SUP + s, pl.ds(off, ln)]],
                        rows_v.at[buf, s, pl.ds(off, ln)],
                        gsem,
                    )
                )
        return cps

    def fire_wb(sc, buf):
        return pltpu.async_copy(
            rows_v.at[buf], out_hbm.at[pl.ds(row_base + sc * SUP, SUP)], wsem
        )

    def half_step(sc_gather, buf_gather):
        g = fire_gathers(sc_gather, buf_gather)
        w = fire_wb(sc_gather - 1, 1 - buf_gather)
        for cp in g:
            cp.wait()
        w.wait()

    for cp in fire_gathers(0, 0):
        cp.wait()

    def body(k, carry):
        half_step(2 * k + 1, 1)
        half_step(2 * k + 2, 0)
        return carry

    lax.fori_loop(0, (NSUP - 2) // 2, body, 0)

    half_step(NSUP - 1, 1)
    fire_wb(NSUP - 1, 1).wait()


def kernel(x, table):
    return _gather_kernel(x, table)


# speedup vs baseline: 1.0022x; 1.0022x over previous
import functools

import jax
import jax.numpy as jnp
from jax import lax
from jax.experimental import pallas as pl
from jax.experimental.pallas import tpu as pltpu
from jax.experimental.pallas import tpu_sc as plsc

B = 4096
S = 200
D = 64
NW = 32
ROWS_PER_W = B // NW
SUP = 2
NSUP = ROWS_PER_W // SUP
SPLITS = ((0, 104), (104, 96))

_mesh = plsc.VectorSubcoreMesh(core_axis_name="c", subcore_axis_name="s")


@functools.partial(
    pl.kernel,
    out_type=jax.ShapeDtypeStruct((B * S, D), jnp.float32),
    mesh=_mesh,
    scratch_types=[
        pltpu.VMEM((ROWS_PER_W, S), jnp.int32),
        pltpu.VMEM((2, SUP * S, D), jnp.float32),
        pltpu.SemaphoreType.DMA,
        pltpu.SemaphoreType.DMA,
    ],
    compiler_params=pltpu.CompilerParams(use_tc_tiling_on_sc=False),
)
def _gather_kernel(x_hbm, table_hbm, out_hbm, idx_v, rows_v, gsem, wsem):
    wid = lax.axis_index("s") * 2 + lax.axis_index("c")
    row_base = wid * ROWS_PER_W
    pltpu.sync_copy(x_hbm.at[pl.ds(row_base, ROWS_PER_W)], idx_v)

    def fire_gathers(sc, buf):
        cps = []
        for s in range(SUP):
            for off, ln in SPLITS:
                cps.append(
                    pltpu.async_copy(
                        table_hbm.at[idx_v.at[sc * SUP + s, pl.ds(off, ln)]],
                        rows_v.at[buf, pl.ds(s * S + off, ln)],
                        gsem,
                    )
                )
        return cps

    def fire_wb(sc, buf):
        row0 = (row_base + sc * SUP) * S
        return pltpu.async_copy(
            rows_v.at[buf], out_hbm.at[pl.ds(row0, SUP * S)], wsem
        )

    def half_step(sc_gather, buf_gather):
        g = fire_gathers(sc_gather, buf_gather)
        w = fire_wb(sc_gather - 1, 1 - buf_gather)
        for cp in g:
            cp.wait()
        w.wait()

    for cp in fire_gathers(0, 0):
        cp.wait()

    def body(k, carry):
        half_step(2 * k + 1, 1)
        half_step(2 * k + 2, 0)
        return carry

    lax.fori_loop(0, (NSUP - 2) // 2, body, 0)

    half_step(NSUP - 1, 1)
    fire_wb(NSUP - 1, 1).wait()


def kernel(x, table):
    return _gather_kernel(x, table).reshape(B, S, D)


# speedup vs baseline: 1.0027x; 1.0005x over previous
import functools

import jax
import jax.numpy as jnp
from jax import lax
from jax.experimental import pallas as pl
from jax.experimental.pallas import tpu as pltpu
from jax.experimental.pallas import tpu_sc as plsc

B = 4096
S = 200
D = 64
NW = 32
ROWS_PER_W = B // NW
SUP = 4
NSUP = ROWS_PER_W // SUP
SPLITS = ((0, 104), (104, 96))

_mesh = plsc.VectorSubcoreMesh(core_axis_name="c", subcore_axis_name="s")


@functools.partial(
    pl.kernel,
    out_type=jax.ShapeDtypeStruct((B * S, D), jnp.float32),
    mesh=_mesh,
    scratch_types=[
        pltpu.VMEM((ROWS_PER_W, S), jnp.int32),
        pltpu.VMEM((2, SUP * S, D), jnp.float32),
        pltpu.SemaphoreType.DMA,
        pltpu.SemaphoreType.DMA,
    ],
    compiler_params=pltpu.CompilerParams(use_tc_tiling_on_sc=False),
)
def _gather_kernel(x_hbm, table_hbm, out_hbm, idx_v, rows_v, gsem, wsem):
    wid = lax.axis_index("s") * 2 + lax.axis_index("c")
    row_base = wid * ROWS_PER_W
    pltpu.sync_copy(x_hbm.at[pl.ds(row_base, ROWS_PER_W)], idx_v)

    def fire_gathers(sc, buf):
        cps = []
        for s in range(SUP):
            for off, ln in SPLITS:
                cps.append(
                    pltpu.async_copy(
                        table_hbm.at[idx_v.at[sc * SUP + s, pl.ds(off, ln)]],
                        rows_v.at[buf, pl.ds(s * S + off, ln)],
                        gsem,
                    )
                )
        return cps

    def fire_wb(sc, buf):
        row0 = (row_base + sc * SUP) * S
        return pltpu.async_copy(
            rows_v.at[buf], out_hbm.at[pl.ds(row0, SUP * S)], wsem
        )

    def half_step(sc_gather, buf_gather):
        g = fire_gathers(sc_gather, buf_gather)
        w = fire_wb(sc_gather - 1, 1 - buf_gather)
        for cp in g:
            cp.wait()
        w.wait()

    for cp in fire_gathers(0, 0):
        cp.wait()

    def body(k, carry):
        half_step(2 * k + 1, 1)
        half_step(2 * k + 2, 0)
        return carry

    lax.fori_loop(0, (NSUP - 2) // 2, body, 0)

    half_step(NSUP - 1, 1)
    fire_wb(NSUP - 1, 1).wait()


def kernel(x, table):
    return _gather_kernel(x, table).reshape(B, S, D)
